# baseline (speedup 1.0000x reference)
import jax
from jax import lax
import jax.numpy as jnp
from jax.experimental import pallas as pl
from jax.experimental.pallas import tpu as pltpu
from jax.experimental.pallas import tpu_sc as plsc

GATE_VALUE = 0.5
TOPK = 2
BLOCK_B = 512

NC, NS, LANES = 2, 16, 16
NW = NC * NS


def _tc_kernel(idx_ref, scores_ref, mask_ref, states_ref):
    idx_ref[...] = jax.lax.broadcasted_iota(jnp.int32, idx_ref.shape, 1)
    scores_ref[...] = jnp.full(scores_ref.shape, GATE_VALUE,
                               dtype=scores_ref.dtype)
    col = jax.lax.broadcasted_iota(jnp.int32, mask_ref.shape, 1)
    mask_ref[...] = col < TOPK
    states_ref[...] = jnp.zeros(states_ref.shape, dtype=states_ref.dtype)


def _sc_gates_kernel(g0_hbm, g1_hbm, g2_hbm, g3_hbm, gate_v, sem):
    rows_per_w = g0_hbm.shape[0] // NW
    wid = lax.axis_index("s") * NC + lax.axis_index("c")

    @pl.loop(0, rows_per_w)
    def _(i):
        gate_v[pl.ds(i, 1), :] = jnp.full((1, LANES), GATE_VALUE, jnp.float32)

    base = wid * rows_per_w
    copies = [
        pltpu.make_async_copy(gate_v, g0_hbm.at[pl.ds(base, rows_per_w)], sem),
        pltpu.make_async_copy(gate_v, g1_hbm.at[pl.ds(base, rows_per_w)], sem),
        pltpu.make_async_copy(gate_v, g2_hbm.at[pl.ds(base, rows_per_w)], sem),
        pltpu.make_async_copy(gate_v, g3_hbm.at[pl.ds(base, rows_per_w)], sem),
    ]
    for c in copies:
        c.start()
    for c in copies:
        c.wait()


def kernel(event, slot_states):
    batch_size, num_slots, slot_dim = slot_states.shape
    kspec = pl.BlockSpec((BLOCK_B, TOPK), lambda i: (i, 0))
    idx, scores, mask, states = pl.pallas_call(
        _tc_kernel,
        grid=(batch_size // BLOCK_B,),
        out_specs=[
            kspec, kspec,
            pl.BlockSpec((BLOCK_B, num_slots), lambda i: (i, 0)),
            pl.BlockSpec((BLOCK_B, TOPK, slot_dim), lambda i: (i, 0, 0)),
        ],
        out_shape=[
            jax.ShapeDtypeStruct((batch_size, TOPK), jnp.int32),
            jax.ShapeDtypeStruct((batch_size, TOPK), jnp.float32),
            jax.ShapeDtypeStruct((batch_size, num_slots), jnp.bool_),
            jax.ShapeDtypeStruct((batch_size, TOPK, slot_dim), jnp.float32),
        ],
    )()

    gshape = jax.ShapeDtypeStruct((batch_size, num_slots), jnp.float32)
    sc_gates = pl.kernel(
        _sc_gates_kernel,
        out_type=[gshape, gshape, gshape, gshape],
        mesh=plsc.VectorSubcoreMesh(core_axis_name="c", subcore_axis_name="s"),
        scratch_types=[
            pltpu.VMEM((batch_size // NW, LANES), jnp.float32),
            pltpu.SemaphoreType.DMA,
        ],
    )
    g0, g1, g2, g3 = sc_gates()
    return (g0, g1, g2, g3, idx, scores, mask, states)


# speedup vs baseline: 1.0004x; 1.0004x over previous
import jax
from jax import lax
import jax.numpy as jnp
from jax.experimental import pallas as pl
from jax.experimental.pallas import tpu as pltpu
from jax.experimental.pallas import tpu_sc as plsc

GATE_VALUE = 0.5
TOPK = 2
BLOCK_B = 512

NC, NS, LANES = 2, 16, 16
NW = NC * NS


def _tc_kernel(idx_ref, scores_ref, mask_ref, states_ref):
    idx_ref[...] = jax.lax.broadcasted_iota(jnp.int32, idx_ref.shape, 1)
    scores_ref[...] = jnp.full(scores_ref.shape, GATE_VALUE,
                               dtype=scores_ref.dtype)
    col = jax.lax.broadcasted_iota(jnp.int32, mask_ref.shape, 1)
    mask_ref[...] = col < TOPK
    states_ref[...] = jnp.zeros(states_ref.shape, dtype=states_ref.dtype)


def _sc_gates_kernel(g0_hbm, g1_hbm, g2_hbm, g3_hbm, gate_v, sem):
    rows_per_w = g0_hbm.shape[0] // NW
    wid = lax.axis_index("s") * NC + lax.axis_index("c")

    @pl.loop(0, rows_per_w)
    def _(i):
        gate_v[pl.ds(i, 1), :] = jnp.full((1, LANES), GATE_VALUE, jnp.float32)

    base = wid * rows_per_w
    copies = [
        pltpu.make_async_copy(gate_v, g0_hbm.at[pl.ds(base, rows_per_w)], sem),
        pltpu.make_async_copy(gate_v, g1_hbm.at[pl.ds(base, rows_per_w)], sem),
        pltpu.make_async_copy(gate_v, g2_hbm.at[pl.ds(base, rows_per_w)], sem),
        pltpu.make_async_copy(gate_v, g3_hbm.at[pl.ds(base, rows_per_w)], sem),
    ]
    for c in copies:
        c.start()
    for c in copies:
        c.wait()


def kernel(event, slot_states):
    batch_size, num_slots, slot_dim = slot_states.shape
    gshape = jax.ShapeDtypeStruct((batch_size, num_slots), jnp.float32)
    sc_gates = pl.kernel(
        _sc_gates_kernel,
        out_type=[gshape, gshape, gshape, gshape],
        mesh=plsc.VectorSubcoreMesh(core_axis_name="c", subcore_axis_name="s"),
        scratch_types=[
            pltpu.VMEM((batch_size // NW, LANES), jnp.float32),
            pltpu.SemaphoreType.DMA,
        ],
    )
    g0, g1, g2, g3 = sc_gates()

    kspec = pl.BlockSpec((BLOCK_B, TOPK), lambda i: (i, 0))
    idx, scores, mask, states = pl.pallas_call(
        _tc_kernel,
        grid=(batch_size // BLOCK_B,),
        out_specs=[
            kspec, kspec,
            pl.BlockSpec((BLOCK_B, num_slots), lambda i: (i, 0)),
            pl.BlockSpec((BLOCK_B, TOPK, slot_dim), lambda i: (i, 0, 0)),
        ],
        out_shape=[
            jax.ShapeDtypeStruct((batch_size, TOPK), jnp.int32),
            jax.ShapeDtypeStruct((batch_size, TOPK), jnp.float32),
            jax.ShapeDtypeStruct((batch_size, num_slots), jnp.bool_),
            jax.ShapeDtypeStruct((batch_size, TOPK, slot_dim), jnp.float32),
        ],
    )()

    return (g0, g1, g2, g3, idx, scores, mask, states)


# speedup vs baseline: 1.7900x; 1.7893x over previous
import jax
import jax.numpy as jnp
from jax.experimental import pallas as pl

GATE_VALUE = 0.5

TOPK = 2
BLOCK_B = 1024


def _fill_kernel(gate_ref, idx_ref, scores_ref, mask_ref, states_ref):
    gate_ref[...] = jnp.full(gate_ref.shape, GATE_VALUE, dtype=gate_ref.dtype)
    idx_ref[...] = jax.lax.broadcasted_iota(jnp.int32, idx_ref.shape, 1)
    scores_ref[...] = jnp.full(scores_ref.shape, GATE_VALUE,
                               dtype=scores_ref.dtype)
    col = jax.lax.broadcasted_iota(jnp.int32, mask_ref.shape, 1)
    mask_ref[...] = col < TOPK
    states_ref[...] = jnp.zeros(states_ref.shape, dtype=states_ref.dtype)


def kernel(event, slot_states):
    batch_size, num_slots, slot_dim = slot_states.shape
    grid = (batch_size // BLOCK_B,)
    gate, idx, scores, mask, states = pl.pallas_call(
        _fill_kernel,
        grid=grid,
        out_specs=[
            pl.BlockSpec((BLOCK_B, num_slots), lambda i: (i, 0)),
            pl.BlockSpec((BLOCK_B, TOPK), lambda i: (i, 0)),
            pl.BlockSpec((BLOCK_B, TOPK), lambda i: (i, 0)),
            pl.BlockSpec((BLOCK_B, num_slots), lambda i: (i, 0)),
            pl.BlockSpec((BLOCK_B, TOPK, slot_dim), lambda i: (i, 0, 0)),
        ],
        out_shape=[
            jax.ShapeDtypeStruct((batch_size, num_slots), jnp.float32),
            jax.ShapeDtypeStruct((batch_size, TOPK), jnp.int32),
            jax.ShapeDtypeStruct((batch_size, TOPK), jnp.float32),
            jax.ShapeDtypeStruct((batch_size, num_slots), jnp.bool_),
            jax.ShapeDtypeStruct((batch_size, TOPK, slot_dim), jnp.float32),
        ],
    )()
    return (gate, gate, gate, gate, idx, scores, mask, states)


# speedup vs baseline: 1.8500x; 1.0335x over previous
import jax
import jax.numpy as jnp
from jax.experimental import pallas as pl

GATE_VALUE = 0.5

TOPK = 2
BLOCK_B = 256


def _fill_kernel(gate_ref, idx_ref, scores_ref, mask_ref, states_ref):
    gate_ref[...] = jnp.full(gate_ref.shape, GATE_VALUE, dtype=gate_ref.dtype)
    idx_ref[...] = jax.lax.broadcasted_iota(jnp.int32, idx_ref.shape, 1)
    scores_ref[...] = jnp.full(scores_ref.shape, GATE_VALUE,
                               dtype=scores_ref.dtype)
    col = jax.lax.broadcasted_iota(jnp.int32, mask_ref.shape, 1)
    mask_ref[...] = col < TOPK
    states_ref[...] = jnp.zeros(states_ref.shape, dtype=states_ref.dtype)


def kernel(event, slot_states):
    batch_size, num_slots, slot_dim = slot_states.shape
    grid = (batch_size // BLOCK_B,)
    gate, idx, scores, mask, states = pl.pallas_call(
        _fill_kernel,
        grid=grid,
        out_specs=[
            pl.BlockSpec((BLOCK_B, num_slots), lambda i: (i, 0)),
            pl.BlockSpec((BLOCK_B, TOPK), lambda i: (i, 0)),
            pl.BlockSpec((BLOCK_B, TOPK), lambda i: (i, 0)),
            pl.BlockSpec((BLOCK_B, num_slots), lambda i: (i, 0)),
            pl.BlockSpec((BLOCK_B, TOPK, slot_dim), lambda i: (i, 0, 0)),
        ],
        out_shape=[
            jax.ShapeDtypeStruct((batch_size, num_slots), jnp.float32),
            jax.ShapeDtypeStruct((batch_size, TOPK), jnp.int32),
            jax.ShapeDtypeStruct((batch_size, TOPK), jnp.float32),
            jax.ShapeDtypeStruct((batch_size, num_slots), jnp.bool_),
            jax.ShapeDtypeStruct((batch_size, TOPK, slot_dim), jnp.float32),
        ],
    )()
    return (gate, gate, gate, gate, idx, scores, mask, states)


# speedup vs baseline: 1.8782x; 1.0152x over previous
import jax
import jax.numpy as jnp
from jax.experimental import pallas as pl
from jax.experimental.pallas import tpu as pltpu

GATE_VALUE = 0.5

TOPK = 2
BLOCK_B = 512


def _fill_kernel(gate_ref, idx_ref, scores_ref, mask_ref, states_ref):
    gate_ref[...] = jnp.full(gate_ref.shape, GATE_VALUE, dtype=gate_ref.dtype)
    idx_ref[...] = jax.lax.broadcasted_iota(jnp.int32, idx_ref.shape, 1)
    scores_ref[...] = jnp.full(scores_ref.shape, GATE_VALUE,
                               dtype=scores_ref.dtype)
    col = jax.lax.broadcasted_iota(jnp.int32, mask_ref.shape, 1)
    mask_ref[...] = col < TOPK
    states_ref[...] = jnp.zeros(states_ref.shape, dtype=states_ref.dtype)


def kernel(event, slot_states):
    batch_size, num_slots, slot_dim = slot_states.shape
    grid = (batch_size // BLOCK_B,)
    gate, idx, scores, mask, states = pl.pallas_call(
        _fill_kernel,
        grid=grid,
        out_specs=[
            pl.BlockSpec((BLOCK_B, num_slots), lambda i: (i, 0)),
            pl.BlockSpec((BLOCK_B, TOPK), lambda i: (i, 0)),
            pl.BlockSpec((BLOCK_B, TOPK), lambda i: (i, 0)),
            pl.BlockSpec((BLOCK_B, num_slots), lambda i: (i, 0)),
            pl.BlockSpec((BLOCK_B, TOPK, slot_dim), lambda i: (i, 0, 0)),
        ],
        compiler_params=pltpu.CompilerParams(
            dimension_semantics=("parallel",)),
        out_shape=[
            jax.ShapeDtypeStruct((batch_size, num_slots), jnp.float32),
            jax.ShapeDtypeStruct((batch_size, TOPK), jnp.int32),
            jax.ShapeDtypeStruct((batch_size, TOPK), jnp.float32),
            jax.ShapeDtypeStruct((batch_size, num_slots), jnp.bool_),
            jax.ShapeDtypeStruct((batch_size, TOPK, slot_dim), jnp.float32),
        ],
    )()
    return (gate, gate, gate, gate, idx, scores, mask, states)
